# baseline (speedup 1.0000x reference)
import functools

import jax
import jax.numpy as jnp
from jax import lax
from jax.experimental import pallas as pl
from jax.experimental.pallas import tpu as pltpu
from jax.experimental.pallas import tpu_sc as plsc

VOCAB = 100000
DIM = 128
BATCH = 1024

V_TILE = 5000
CHUNK = 1000
N_STEPS = VOCAB // V_TILE
N_CHUNKS = V_TILE // CHUNK


@functools.cache
def _make_gather():
    info = plsc.get_sparse_core_info()
    nc, ns = info.num_cores, info.num_subcores
    nw = nc * ns
    b_per_w = BATCH // nw
    mesh = plsc.VectorSubcoreMesh(core_axis_name="c", subcore_axis_name="s")

    @functools.partial(
        pl.kernel,
        mesh=mesh,
        out_type=jax.ShapeDtypeStruct((BATCH, DIM), jnp.float32),
        scratch_types=[
            pltpu.VMEM((b_per_w,), jnp.int32),
            pltpu.VMEM((b_per_w, DIM), jnp.float32),
            pltpu.SemaphoreType.DMA,
        ],
    )
    def gather(idx_hbm, table_hbm, out_hbm, idx_v, rows_v, sem):
        wid = lax.axis_index("s") * nc + lax.axis_index("c")
        base = wid * b_per_w
        pltpu.sync_copy(idx_hbm.at[pl.ds(base, b_per_w)], idx_v)
        pltpu.async_copy(table_hbm.at[idx_v], rows_v, sem).wait()
        pltpu.sync_copy(rows_v, out_hbm.at[pl.ds(base, b_per_w)])

    return gather


def _proj_kernel(emb_ref, w_ref, b_ref, out_ref, embt_ref, bcol_ref,
                 obuf_ref, sems):
    i = pl.program_id(0)

    @pl.when(i == 0)
    def _():
        embt_ref[...] = emb_ref[...].T

    bcol_ref[...] = b_ref[0].T

    for sub in range(N_CHUNKS):
        def _copy(slot):
            return pltpu.make_async_copy(
                obuf_ref.at[slot],
                out_ref.at[pl.ds(i * V_TILE + slot * CHUNK, CHUNK), :],
                sems.at[slot],
            )

        @pl.when(i > 0)
        def _():
            _copy(sub).wait()

        obuf_ref[sub] = lax.dot_general(
            w_ref[pl.ds(sub * CHUNK, CHUNK), :], embt_ref[...],
            dimension_numbers=(((1,), (0,)), ((), ())),
            preferred_element_type=jnp.float32,
        ) + bcol_ref[pl.ds(sub * CHUNK, CHUNK), :]
        _copy(sub).start()

    @pl.when(i == N_STEPS - 1)
    def _():
        for sub in range(N_CHUNKS):
            pltpu.make_async_copy(
                obuf_ref.at[sub],
                out_ref.at[pl.ds(i * V_TILE + sub * CHUNK, CHUNK), :],
                sems.at[sub],
            ).wait()


def _project(emb, lin_w, lin_b3d):
    return pl.pallas_call(
        _proj_kernel,
        grid=(N_STEPS,),
        in_specs=[
            pl.BlockSpec((BATCH, DIM), lambda i: (0, 0)),
            pl.BlockSpec((V_TILE, DIM), lambda i: (i, 0)),
            pl.BlockSpec((1, 1, V_TILE), lambda i: (i, 0, 0)),
        ],
        out_specs=pl.BlockSpec(memory_space=pl.ANY),
        out_shape=jax.ShapeDtypeStruct((VOCAB, BATCH), jnp.float32),
        scratch_shapes=[
            pltpu.VMEM((DIM, BATCH), jnp.float32),
            pltpu.VMEM((V_TILE, 1), jnp.float32),
            pltpu.VMEM((N_CHUNKS, CHUNK, BATCH), jnp.float32),
            pltpu.SemaphoreType.DMA((N_CHUNKS,)),
        ],
    )(emb, lin_w, lin_b3d)


def kernel(input_ids, emb_table, lin_w, lin_b):
    emb = _make_gather()(input_ids, emb_table)
    out_t = _project(emb, lin_w, lin_b.reshape(N_STEPS, 1, V_TILE))
    return out_t.T


# speedup vs baseline: 1.0038x; 1.0038x over previous
import functools

import jax
import jax.numpy as jnp
from jax import lax
from jax.experimental import pallas as pl
from jax.experimental.pallas import tpu as pltpu
from jax.experimental.pallas import tpu_sc as plsc

VOCAB = 100000
DIM = 128
BATCH = 1024

V_TILE = 4096
CHUNK = 1024
N_STEPS = -(-VOCAB // V_TILE)
N_CHUNKS = V_TILE // CHUNK
TAIL = VOCAB - (N_STEPS - 1) * V_TILE
TAIL_CHUNKS = [CHUNK, TAIL - CHUNK]


@functools.cache
def _make_gather():
    info = plsc.get_sparse_core_info()
    nc, ns = info.num_cores, info.num_subcores
    nw = nc * ns
    b_per_w = BATCH // nw
    mesh = plsc.VectorSubcoreMesh(core_axis_name="c", subcore_axis_name="s")

    @functools.partial(
        pl.kernel,
        mesh=mesh,
        out_type=jax.ShapeDtypeStruct((BATCH, DIM), jnp.float32),
        scratch_types=[
            pltpu.VMEM((b_per_w,), jnp.int32),
            pltpu.VMEM((b_per_w, DIM), jnp.float32),
            pltpu.SemaphoreType.DMA,
        ],
    )
    def gather(idx_hbm, table_hbm, out_hbm, idx_v, rows_v, sem):
        wid = lax.axis_index("s") * nc + lax.axis_index("c")
        base = wid * b_per_w
        pltpu.sync_copy(idx_hbm.at[pl.ds(base, b_per_w)], idx_v)
        pltpu.async_copy(table_hbm.at[idx_v], rows_v, sem).wait()
        pltpu.sync_copy(rows_v, out_hbm.at[pl.ds(base, b_per_w)])

    return gather


def _proj_kernel(emb_ref, w_ref, b_ref, out_ref, embt_ref, bcol_ref,
                 obuf_ref, sems):
    i = pl.program_id(0)

    @pl.when(i == 0)
    def _():
        embt_ref[...] = emb_ref[...].T

    bcol_ref[...] = b_ref[...].reshape(V_TILE, 1)

    def _chunk_copy(slot, base, rows):
        return pltpu.make_async_copy(
            obuf_ref.at[slot, pl.ds(0, rows)],
            out_ref.at[pl.ds(base, rows), :],
            sems.at[slot],
        )

    def _do_chunk(sub, rows):
        @pl.when(i > 0)
        def _():
            _chunk_copy(sub, 0, CHUNK).wait()

        obuf_ref[sub, pl.ds(0, rows)] = lax.dot_general(
            w_ref[pl.ds(sub * CHUNK, rows), :], embt_ref[...],
            dimension_numbers=(((1,), (0,)), ((), ())),
            preferred_element_type=jnp.float32,
        ) + bcol_ref[pl.ds(sub * CHUNK, rows), :]
        _chunk_copy(sub, i * V_TILE + sub * CHUNK, rows).start()

    @pl.when(i < N_STEPS - 1)
    def _():
        for sub in range(N_CHUNKS):
            _do_chunk(sub, CHUNK)

    @pl.when(i == N_STEPS - 1)
    def _():
        for sub, rows in enumerate(TAIL_CHUNKS):
            _do_chunk(sub, rows)
        for sub in range(N_CHUNKS):
            rows = TAIL_CHUNKS[sub] if sub < len(TAIL_CHUNKS) else CHUNK
            _chunk_copy(sub, 0, rows).wait()


def _project(emb, lin_w, lin_b1d):
    return pl.pallas_call(
        _proj_kernel,
        grid=(N_STEPS,),
        in_specs=[
            pl.BlockSpec((BATCH, DIM), lambda i: (0, 0)),
            pl.BlockSpec((V_TILE, DIM), lambda i: (i, 0)),
            pl.BlockSpec((V_TILE,), lambda i: (i,)),
        ],
        out_specs=pl.BlockSpec(memory_space=pl.ANY),
        out_shape=jax.ShapeDtypeStruct((VOCAB, BATCH), jnp.float32),
        scratch_shapes=[
            pltpu.VMEM((DIM, BATCH), jnp.float32),
            pltpu.VMEM((V_TILE, 1), jnp.float32),
            pltpu.VMEM((N_CHUNKS, CHUNK, BATCH), jnp.float32),
            pltpu.SemaphoreType.DMA((N_CHUNKS,)),
        ],
    )(emb, lin_w, lin_b1d)


def kernel(input_ids, emb_table, lin_w, lin_b):
    emb = _make_gather()(input_ids, emb_table)
    out_t = _project(emb, lin_w, lin_b)
    return out_t.T
